# baseline (speedup 1.0000x reference)
import jax
import jax.numpy as jnp
from jax import lax
from jax.experimental import pallas as pl
from jax.experimental.pallas import tpu as pltpu
from jax.experimental.pallas import tpu_sc as plsc

F = 26
V = 100000
D = 32
B = 16384

NW = 32
NTASK = F * D
PER_W = NTASK // NW
CHUNK = 4096
NCHUNK = B // CHUNK
L = 16
UNROLL = 8

PIECES = ((0, 25088), (25088, 24960), (50048, 24960), (75008, 24960))
TS = 99968
VP = 100096


def _body(tab_hbm, tail_hbm, x_hbm, out_hbm, row_v, idx_v, out_v, sem_r, sem_w):
    cid = lax.axis_index("c")
    sid = lax.axis_index("s")
    wid = sid * 2 + cid

    def coords(task):
        return task // D, task % D

    def fire_row(f, c):
        for off, ln in PIECES:
            pltpu.async_copy(
                tab_hbm.at[f, c, pl.ds(off, ln)], row_v.at[pl.ds(off, ln)], sem_r
            )
        pltpu.async_copy(tail_hbm.at[f, c], row_v.at[pl.ds(TS, 128)], sem_r)

    def wait_row(f, c):
        for off, ln in PIECES:
            pltpu.make_async_copy(
                tab_hbm.at[f, c, pl.ds(off, ln)], row_v.at[pl.ds(off, ln)], sem_r
            ).wait()
        pltpu.make_async_copy(
            tail_hbm.at[f, c], row_v.at[pl.ds(TS, 128)], sem_r
        ).wait()

    def fire_write(task, k):
        pltpu.async_copy(
            out_v.at[k % 2],
            out_hbm.at[task, pl.ds(k * CHUNK, CHUNK)],
            sem_w,
        )

    def wait_write(task, k):
        pltpu.make_async_copy(
            out_v.at[k % 2],
            out_hbm.at[task, pl.ds(k * CHUNK, CHUNK)],
            sem_w,
        ).wait()

    f0, c0 = coords(wid * PER_W)
    fire_row(f0, c0)

    def task_body(j, fprev):
        task = wid * PER_W + j
        f, c = coords(task)
        wait_row(f, c)

        @pl.when(f != fprev)
        def _():
            pltpu.sync_copy(x_hbm.at[f], idx_v)

        for k in range(NCHUNK):
            if k >= 2:
                wait_write(task, k - 2)
            else:

                @pl.when(j > 0)
                def _():
                    wait_write(task, k + 2)

            def gather_body(g, carry2):
                base = g * (L * UNROLL)
                for u in range(UNROLL):
                    off = base + u * L
                    iv = idx_v[pl.ds(k * CHUNK + off, L)]
                    out_v[k % 2, pl.ds(off, L)] = plsc.load_gather(row_v, [iv])
                return carry2

            lax.fori_loop(0, CHUNK // (L * UNROLL), gather_body, 0)

            if k == NCHUNK - 1:
                @pl.when(j + 1 < PER_W)
                def _():
                    fn, cn = coords(task + 1)
                    fire_row(fn, cn)

            fire_write(task, k)

        return f

    lax.fori_loop(0, PER_W, task_body, jnp.int32(-1))
    last = wid * PER_W + PER_W - 1
    wait_write(last, NCHUNK - 2)
    wait_write(last, NCHUNK - 1)


@jax.jit
def _run(x_t, tab_t, tail_t):
    mesh = plsc.VectorSubcoreMesh(core_axis_name="c", subcore_axis_name="s")
    kfn = pl.kernel(
        _body,
        mesh=mesh,
        out_type=jax.ShapeDtypeStruct((NTASK, B), jnp.float32),
        scratch_types=[
            pltpu.VMEM((VP,), jnp.float32),
            pltpu.VMEM((B,), jnp.int32),
            pltpu.VMEM((2, CHUNK), jnp.float32),
            pltpu.SemaphoreType.DMA,
            pltpu.SemaphoreType.DMA,
        ],
        compiler_params=pltpu.CompilerParams(
            use_tc_tiling_on_sc=True, needs_layout_passes=False
        ),
    )
    return kfn(tab_t, tail_t, x_t)


def kernel(x, tables):
    x_t = x.astype(jnp.int32).T
    tab_t = jnp.transpose(tables, (0, 2, 1))
    tail_t = jnp.pad(tab_t[:, :, TS:], ((0, 0), (0, 0), (0, 128 - (V - TS))))
    out_t = _run(x_t, tab_t, tail_t)
    return out_t.T


# speedup vs baseline: 1.4561x; 1.4561x over previous
import jax
import jax.numpy as jnp
from jax import lax
from jax.experimental import pallas as pl
from jax.experimental.pallas import tpu as pltpu
from jax.experimental.pallas import tpu_sc as plsc

F = 26
V = 100000
D = 32
B = 16384

NW = 32
NTASK = F * D
PER_W = NTASK // NW
CHUNK = 8192
NCHUNK = B // CHUNK
L = 16
UNROLL = 8

PIECES = ((0, 25088), (25088, 24960), (50048, 24960), (75008, 24960))
TS = 99968
VP = 100096


def _body(tab_hbm, tail_hbm, x_hbm, out_hbm, row_v, idx_v, out_v, sem_r):
    cid = lax.axis_index("c")
    sid = lax.axis_index("s")
    wid = sid * 2 + cid

    def coords(task):
        return task // D, task % D

    def fire_row(f, c):
        for off, ln in PIECES:
            pltpu.async_copy(
                tab_hbm.at[f, c, pl.ds(off, ln)], row_v.at[pl.ds(off, ln)], sem_r
            )
        pltpu.async_copy(tail_hbm.at[f, c], row_v.at[pl.ds(TS, 128)], sem_r)

    def wait_row(f, c):
        for off, ln in PIECES:
            pltpu.make_async_copy(
                tab_hbm.at[f, c, pl.ds(off, ln)], row_v.at[pl.ds(off, ln)], sem_r
            ).wait()
        pltpu.make_async_copy(
            tail_hbm.at[f, c], row_v.at[pl.ds(TS, 128)], sem_r
        ).wait()

    f0, c0 = coords(wid * PER_W)
    fire_row(f0, c0)

    def task_body(j, fprev):
        task = wid * PER_W + j
        f, c = coords(task)
        wait_row(f, c)

        @pl.when(f != fprev)
        def _():
            pltpu.sync_copy(x_hbm.at[f], idx_v)

        for k in range(NCHUNK):

            def gather_body(g, carry2):
                base = g * (L * UNROLL)
                for u in range(UNROLL):
                    off = base + u * L
                    iv = idx_v[pl.ds(k * CHUNK + off, L)]
                    out_v[pl.ds(off, L)] = plsc.load_gather(row_v, [iv])
                return carry2

            lax.fori_loop(0, CHUNK // (L * UNROLL), gather_body, 0)

            if k == NCHUNK - 1:
                @pl.when(j + 1 < PER_W)
                def _():
                    fn, cn = coords(task + 1)
                    fire_row(fn, cn)

            pltpu.sync_copy(out_v, out_hbm.at[task, pl.ds(k * CHUNK, CHUNK)])

        return f

    lax.fori_loop(0, PER_W, task_body, jnp.int32(-1))


@jax.jit
def _run(x_t, tab_t, tail_t):
    mesh = plsc.VectorSubcoreMesh(core_axis_name="c", subcore_axis_name="s")
    kfn = pl.kernel(
        _body,
        mesh=mesh,
        out_type=jax.ShapeDtypeStruct((NTASK, B), jnp.float32),
        scratch_types=[
            pltpu.VMEM((VP,), jnp.float32),
            pltpu.VMEM((B,), jnp.int32),
            pltpu.VMEM((CHUNK,), jnp.float32),
            pltpu.SemaphoreType.DMA,
        ],
        compiler_params=pltpu.CompilerParams(
            use_tc_tiling_on_sc=True, needs_layout_passes=False
        ),
    )
    return kfn(tab_t, tail_t, x_t)


def kernel(x, tables):
    x_t = x.astype(jnp.int32).T
    tab_t = jnp.transpose(tables, (0, 2, 1))
    tail_t = jnp.pad(tab_t[:, :, TS:], ((0, 0), (0, 0), (0, 128 - (V - TS))))
    out_t = _run(x_t, tab_t, tail_t)
    return out_t.T
